# baseline (speedup 1.0000x reference)
import functools

import jax
import jax.numpy as jnp
from jax import lax
from jax.experimental import pallas as pl
from jax.experimental.pallas import tpu as pltpu
from jax.experimental.pallas import tpu_sc as plsc

N = 10000
E = 160000
D = 256
H = 256
DH = 128
C = 128
NSUB = 16
NCORE = 2
NK = 80
EPS = NK * C
EP = EPS * NSUB
NP = 10240
ROWS_PER_SUB = NP // NSUB


def _dense_a_body(x_ref, wp_ref, bp_ref, whh_ref, bhh_ref,
                  hvlo_ref, hvhi_ref, gh_ref):
    x = x_ref[...]
    hv = lax.dot_general(x, wp_ref[...], (((1,), (1,)), ((), ())),
                         preferred_element_type=jnp.float32) + bp_ref[...]
    hvlo_ref[...] = hv[:, :DH]
    hvhi_ref[...] = hv[:, DH:]
    gh_ref[...] = lax.dot_general(x, whh_ref[...], (((1,), (1,)), ((), ())),
                                  preferred_element_type=jnp.float32) + bhh_ref[...]


def _dense_a(x, wp, bp, whh, bhh):
    bn = 1000
    grid = (N // bn,)
    return pl.pallas_call(
        _dense_a_body,
        grid=grid,
        in_specs=[
            pl.BlockSpec((bn, D), lambda i: (i, 0)),
            pl.BlockSpec((H, D), lambda i: (0, 0)),
            pl.BlockSpec((1, H), lambda i: (0, 0)),
            pl.BlockSpec((3 * D, D), lambda i: (0, 0)),
            pl.BlockSpec((1, 3 * D), lambda i: (0, 0)),
        ],
        out_specs=[
            pl.BlockSpec((bn, DH), lambda i: (i, 0)),
            pl.BlockSpec((bn, DH), lambda i: (i, 0)),
            pl.BlockSpec((bn, 3 * D), lambda i: (i, 0)),
        ],
        out_shape=[
            jax.ShapeDtypeStruct((N, DH), jnp.float32),
            jax.ShapeDtypeStruct((N, DH), jnp.float32),
            jax.ShapeDtypeStruct((N, 3 * D), jnp.float32),
        ],
    )(x, wp, bp, whh, bhh)


def _sc_body(src_hbm, dst_hbm, lg_hbm, hvlo_hbm, hvhi_hbm,
             clo_hbm, chi_hbm, ssum_hbm,
             srcv, wv, dstv, rows0, rows1, csh, ssh,
             sg0, sg1, ss0, ss1, sw0, sw1, sd0, sd1, se0, se1, se2, se3):
    cid = lax.axis_index("c")
    sid = lax.axis_index("s")
    zero16 = jnp.zeros((16,), jnp.float32)
    base_e = sid * EPS

    pltpu.sync_copy(lg_hbm.at[pl.ds(base_e, EPS)], wv)

    def _expb(i, carry):
        wv[pl.ds(16 * i, 16)] = jnp.exp(wv[pl.ds(16 * i, 16)])
        return carry
    lax.fori_loop(0, EPS // 16, _expb, 0)

    def _zrow(i, carry):
        for j in range(8):
            rows0[i, pl.ds(16 * j, 16)] = zero16
        return carry
    lax.fori_loop(0, C, _zrow, 0)

    base_n = sid * ROWS_PER_SUB
    for m in range(5):
        pltpu.sync_copy(rows0, csh.at[pl.ds(base_n + C * m, C)])
        pltpu.sync_copy(rows0.at[0], ssh.at[pl.ds(base_n + C * m, C)])

    plsc.subcore_barrier()

    def _gather(t, idx_b, rows_b, sem_b):
        @pl.when(cid == 0)
        def _():
            pltpu.async_copy(hvlo_hbm.at[idx_b], rows_b, sem_b)

        @pl.when(cid == 1)
        def _():
            pltpu.async_copy(hvhi_hbm.at[idx_b], rows_b, sem_b)

    def _wait_gather(t, idx_b, rows_b, sem_b):
        @pl.when(cid == 0)
        def _():
            pltpu.make_async_copy(hvlo_hbm.at[idx_b], rows_b, sem_b).wait()

        @pl.when(cid == 1)
        def _():
            pltpu.make_async_copy(hvhi_hbm.at[idx_b], rows_b, sem_b).wait()

    def _issue_scatter(t, rows_b, dst_b, sem_c, sem_w):
        pltpu.async_copy(rows_b, csh.at[dst_b], sem_c, add=True)

        @pl.when(cid == 0)
        def _():
            pltpu.async_copy(wv.at[pl.ds(C * t, C)], ssh.at[dst_b], sem_w,
                             add=True)

    def _wait_scatter(t, rows_b, dst_b, sem_c, sem_w):
        pltpu.make_async_copy(rows_b, csh.at[dst_b], sem_c).wait()

        @pl.when(cid == 0)
        def _():
            pltpu.make_async_copy(wv.at[pl.ds(C * t, C)], ssh.at[dst_b],
                                  sem_w).wait()

    def _src_slice(t):
        return src_hbm.at[pl.ds(base_e + C * t, C)]

    def _dst_slice(t):
        return dst_hbm.at[pl.ds(base_e + C * t, C)]

    rows_bufs = (rows0, rows1)
    sg = (sg0, sg1)
    ss = (ss0, ss1)
    sw = (sw0, sw1)
    sd = (sd0, sd1)
    se = (se0, se1, se2, se3)

    pltpu.sync_copy(_src_slice(0), srcv.at[0])
    pltpu.sync_copy(_dst_slice(0), dstv.at[0])
    _gather(0, srcv.at[0], rows0, sg0)
    pltpu.async_copy(_src_slice(1), srcv.at[1], se[1])

    def _quad(p, carry):
        for q in range(4):
            t = 4 * p + q
            b = q % 2
            rows_b, rows_o = rows_bufs[b], rows_bufs[1 - b]

            @pl.when(t > 0)
            def _():
                _wait_scatter(t - 1, rows_o, dstv.at[1 - b], ss[1 - b],
                              sw[1 - b])

            @pl.when(t < NK - 1)
            def _():
                pltpu.make_async_copy(_src_slice(t + 1),
                                      srcv.at[(q + 1) % 4], se[(q + 1) % 4]
                                      ).wait()
                _gather(t + 1, srcv.at[(q + 1) % 4], rows_o, sg[1 - b])

            @pl.when(t < NK - 2)
            def _():
                pltpu.async_copy(_src_slice(t + 2), srcv.at[(q + 2) % 4],
                                 se[(q + 2) % 4])

            @pl.when(t < NK - 1)
            def _():
                pltpu.async_copy(_dst_slice(t + 1), dstv.at[1 - b],
                                 sd[1 - b])

            _wait_gather(t, srcv.at[q % 4], rows_b, sg[b])

            def _scaleg(g, c2):
                wg = wv[pl.ds(C * t + 16 * g, 16)]
                for j in range(16):
                    splat = jnp.full((16,), wg[j], jnp.float32)
                    i = 16 * g + j
                    for m in range(8):
                        rows_b[i, pl.ds(16 * m, 16)] = (
                            rows_b[i, pl.ds(16 * m, 16)] * splat)
                return c2
            lax.fori_loop(0, C // 16, _scaleg, 0)

            @pl.when(t > 0)
            def _():
                pltpu.make_async_copy(_dst_slice(t), dstv.at[b], sd[b]).wait()

            _issue_scatter(t, rows_b, dstv.at[b], ss[b], sw[b])
        return carry

    lax.fori_loop(0, NK // 4, _quad, 0)

    _wait_scatter(NK - 1, rows1, dstv.at[1], ss1, sw1)

    plsc.subcore_barrier()

    @pl.when(cid == 0)
    def _():
        pltpu.sync_copy(csh.at[pl.ds(base_n, ROWS_PER_SUB)],
                        clo_hbm.at[pl.ds(base_n, ROWS_PER_SUB)])

    @pl.when(cid == 1)
    def _():
        pltpu.sync_copy(csh.at[pl.ds(base_n, ROWS_PER_SUB)],
                        chi_hbm.at[pl.ds(base_n, ROWS_PER_SUB)])

    @pl.when(cid == 0)
    def _():
        pltpu.sync_copy(ssh.at[pl.ds(base_n, ROWS_PER_SUB)],
                        ssum_hbm.at[pl.ds(base_n, ROWS_PER_SUB)])


def _sc_call(src, dst, logits, hvlo, hvhi):
    mesh = plsc.VectorSubcoreMesh(core_axis_name="c", subcore_axis_name="s",
                                  num_cores=NCORE, num_subcores=NSUB)
    f = pl.kernel(
        _sc_body,
        out_type=[
            jax.ShapeDtypeStruct((NP, DH), jnp.float32),
            jax.ShapeDtypeStruct((NP, DH), jnp.float32),
            jax.ShapeDtypeStruct((NP,), jnp.float32),
        ],
        mesh=mesh,
        scratch_types=[
            pltpu.VMEM((4, C), jnp.int32),
            pltpu.VMEM((EPS,), jnp.float32),
            pltpu.VMEM((2, C), jnp.int32),
            pltpu.VMEM((C, DH), jnp.float32),
            pltpu.VMEM((C, DH), jnp.float32),
            pltpu.VMEM_SHARED((NP, DH), jnp.float32),
            pltpu.VMEM_SHARED((NP,), jnp.float32),
        ] + [pltpu.SemaphoreType.DMA] * 12,
    )
    return f(src, dst, logits, hvlo, hvhi)


def _dense_b_body(clo_ref, chi_ref, ssum_ref, gh_ref, h_ref, wih_ref, bih_ref,
                  out_ref):
    s = ssum_ref[...]
    s = jnp.where(s > 0.0, s, 1.0)
    c = jnp.concatenate([clo_ref[...], chi_ref[...]], axis=1) / s
    ctx = jnp.where(c > 0.0, c, jnp.exp(jnp.minimum(c, 0.0)) - 1.0)
    gi = lax.dot_general(ctx.astype(jnp.bfloat16), wih_ref[...],
                         (((1,), (1,)), ((), ())),
                         preferred_element_type=jnp.float32) + bih_ref[...]
    gh = gh_ref[...]
    h = h_ref[...]
    r = jax.nn.sigmoid(gi[:, :D] + gh[:, :D])
    z = jax.nn.sigmoid(gi[:, D:2 * D] + gh[:, D:2 * D])
    n = jnp.tanh(gi[:, 2 * D:] + r * gh[:, 2 * D:])
    out_ref[...] = jnp.maximum((1.0 - z) * n + z * h, 0.0)


def _dense_b(clo, chi, ssum2, gh, h, wih, bih):
    bn = 1000
    grid = (N // bn,)
    return pl.pallas_call(
        _dense_b_body,
        grid=grid,
        in_specs=[
            pl.BlockSpec((bn, DH), lambda i: (i, 0)),
            pl.BlockSpec((bn, DH), lambda i: (i, 0)),
            pl.BlockSpec((bn, 1), lambda i: (i, 0)),
            pl.BlockSpec((bn, 3 * D), lambda i: (i, 0)),
            pl.BlockSpec((bn, D), lambda i: (i, 0)),
            pl.BlockSpec((3 * D, D), lambda i: (0, 0)),
            pl.BlockSpec((1, 3 * D), lambda i: (0, 0)),
        ],
        out_specs=pl.BlockSpec((bn, D), lambda i: (i, 0)),
        out_shape=jax.ShapeDtypeStruct((N, D), jnp.float32),
    )(clo, chi, ssum2, gh, h, wih, bih)


def kernel(edge_index, edge_logits, node_feats, W_p, b_p, W_ih, W_hh, b_ih, b_hh):
    pad = EP - E
    pad_iota = jnp.arange(pad, dtype=jnp.int32)
    src = jnp.concatenate([edge_index[0], pad_iota % N])
    dst = jnp.concatenate([edge_index[1], N + pad_iota % (NP - N)])
    logits = jnp.concatenate(
        [edge_logits[:, 0], jnp.full((pad,), -jnp.inf, jnp.float32)])
    hvlo, hvhi, gh = _dense_a(node_feats.astype(jnp.bfloat16),
                              W_p.astype(jnp.bfloat16), b_p.reshape(1, H),
                              W_hh.astype(jnp.bfloat16),
                              b_hh.reshape(1, 3 * D))
    clo, chi, ssum = _sc_call(src, dst, logits, hvlo, hvhi)
    return _dense_b(clo, chi, ssum.reshape(NP, 1), gh, node_feats,
                    W_ih.astype(jnp.bfloat16), b_ih.reshape(1, 3 * D))


# speedup vs baseline: 1.0649x; 1.0649x over previous
import functools

import jax
import jax.numpy as jnp
from jax import lax
from jax.experimental import pallas as pl
from jax.experimental.pallas import tpu as pltpu
from jax.experimental.pallas import tpu_sc as plsc

N = 10000
E = 160000
D = 256
H = 256
DH = 128
C = 128
NSUB = 16
NCORE = 2
NK = 80
EPS = NK * C
EP = EPS * NSUB
NP = 10240
ROWS_PER_SUB = NP // NSUB


def _dense_a1_body(x_ref, wp_ref, bp_ref, hvlo_ref, hvhi_ref):
    x = x_ref[...]
    hv = lax.dot_general(x, wp_ref[...], (((1,), (1,)), ((), ())),
                         preferred_element_type=jnp.float32) + bp_ref[...]
    hvlo_ref[...] = hv[:, :DH]
    hvhi_ref[...] = hv[:, DH:]


def _dense_a1(x, wp, bp):
    bn = 1000
    grid = (N // bn,)
    return pl.pallas_call(
        _dense_a1_body,
        grid=grid,
        in_specs=[
            pl.BlockSpec((bn, D), lambda i: (i, 0)),
            pl.BlockSpec((H, D), lambda i: (0, 0)),
            pl.BlockSpec((1, H), lambda i: (0, 0)),
        ],
        out_specs=[
            pl.BlockSpec((bn, DH), lambda i: (i, 0)),
            pl.BlockSpec((bn, DH), lambda i: (i, 0)),
        ],
        out_shape=[
            jax.ShapeDtypeStruct((N, DH), jnp.float32),
            jax.ShapeDtypeStruct((N, DH), jnp.float32),
        ],
    )(x, wp, bp)


def _dense_a2_body(x_ref, whh_ref, bhh_ref, gh_ref):
    gh_ref[...] = lax.dot_general(x_ref[...], whh_ref[...],
                                  (((1,), (1,)), ((), ())),
                                  preferred_element_type=jnp.float32
                                  ) + bhh_ref[...]


def _dense_a2(x, whh, bhh):
    bn = 1000
    grid = (N // bn,)
    return pl.pallas_call(
        _dense_a2_body,
        grid=grid,
        in_specs=[
            pl.BlockSpec((bn, D), lambda i: (i, 0)),
            pl.BlockSpec((3 * D, D), lambda i: (0, 0)),
            pl.BlockSpec((1, 3 * D), lambda i: (0, 0)),
        ],
        out_specs=pl.BlockSpec((bn, 3 * D), lambda i: (i, 0)),
        out_shape=jax.ShapeDtypeStruct((N, 3 * D), jnp.float32),
    )(x, whh, bhh)


def _sc_body(src_hbm, dst_hbm, lg_hbm, hvlo_hbm, hvhi_hbm,
             clo_hbm, chi_hbm, ssum_hbm,
             srcv, wv, dstv, rows0, rows1, csh, ssh,
             sg0, sg1, ss0, ss1, sw0, sw1, sd0, sd1, se0, se1, se2, se3):
    cid = lax.axis_index("c")
    sid = lax.axis_index("s")
    zero16 = jnp.zeros((16,), jnp.float32)
    base_e = sid * EPS

    pltpu.sync_copy(lg_hbm.at[pl.ds(base_e, EPS)], wv)

    def _expb(i, carry):
        wv[pl.ds(16 * i, 16)] = jnp.exp(wv[pl.ds(16 * i, 16)])
        return carry
    lax.fori_loop(0, EPS // 16, _expb, 0)

    def _zrow(i, carry):
        for j in range(8):
            rows0[i, pl.ds(16 * j, 16)] = zero16
        return carry
    lax.fori_loop(0, C, _zrow, 0)

    base_n = sid * ROWS_PER_SUB
    for m in range(5):
        pltpu.sync_copy(rows0, csh.at[pl.ds(base_n + C * m, C)])
        pltpu.sync_copy(rows0.at[0], ssh.at[pl.ds(base_n + C * m, C)])

    plsc.subcore_barrier()

    def _gather(t, idx_b, rows_b, sem_b):
        @pl.when(cid == 0)
        def _():
            pltpu.async_copy(hvlo_hbm.at[idx_b], rows_b, sem_b)

        @pl.when(cid == 1)
        def _():
            pltpu.async_copy(hvhi_hbm.at[idx_b], rows_b, sem_b)

    def _wait_gather(t, idx_b, rows_b, sem_b):
        @pl.when(cid == 0)
        def _():
            pltpu.make_async_copy(hvlo_hbm.at[idx_b], rows_b, sem_b).wait()

        @pl.when(cid == 1)
        def _():
            pltpu.make_async_copy(hvhi_hbm.at[idx_b], rows_b, sem_b).wait()

    def _issue_scatter(t, rows_b, dst_b, sem_c, sem_w):
        pltpu.async_copy(rows_b, csh.at[dst_b], sem_c, add=True)

        @pl.when(cid == 0)
        def _():
            pltpu.async_copy(wv.at[pl.ds(C * t, C)], ssh.at[dst_b], sem_w,
                             add=True)

    def _wait_scatter(t, rows_b, dst_b, sem_c, sem_w):
        pltpu.make_async_copy(rows_b, csh.at[dst_b], sem_c).wait()

        @pl.when(cid == 0)
        def _():
            pltpu.make_async_copy(wv.at[pl.ds(C * t, C)], ssh.at[dst_b],
                                  sem_w).wait()

    def _src_slice(t):
        return src_hbm.at[pl.ds(base_e + C * t, C)]

    def _dst_slice(t):
        return dst_hbm.at[pl.ds(base_e + C * t, C)]

    rows_bufs = (rows0, rows1)
    sg = (sg0, sg1)
    ss = (ss0, ss1)
    sw = (sw0, sw1)
    sd = (sd0, sd1)
    se = (se0, se1, se2, se3)

    pltpu.sync_copy(_src_slice(0), srcv.at[0])
    pltpu.sync_copy(_dst_slice(0), dstv.at[0])
    _gather(0, srcv.at[0], rows0, sg0)
    pltpu.async_copy(_src_slice(1), srcv.at[1], se[1])

    def _quad(p, carry):
        for q in range(4):
            t = 4 * p + q
            b = q % 2
            rows_b, rows_o = rows_bufs[b], rows_bufs[1 - b]

            @pl.when(t > 0)
            def _():
                _wait_scatter(t - 1, rows_o, dstv.at[1 - b], ss[1 - b],
                              sw[1 - b])

            @pl.when(t < NK - 1)
            def _():
                pltpu.make_async_copy(_src_slice(t + 1),
                                      srcv.at[(q + 1) % 4], se[(q + 1) % 4]
                                      ).wait()
                _gather(t + 1, srcv.at[(q + 1) % 4], rows_o, sg[1 - b])

            @pl.when(t < NK - 2)
            def _():
                pltpu.async_copy(_src_slice(t + 2), srcv.at[(q + 2) % 4],
                                 se[(q + 2) % 4])

            @pl.when(t < NK - 1)
            def _():
                pltpu.async_copy(_dst_slice(t + 1), dstv.at[1 - b],
                                 sd[1 - b])

            _wait_gather(t, srcv.at[q % 4], rows_b, sg[b])

            @plsc.parallel_loop(0, C // 16, unroll=2)
            def _scaleg(g):
                wg = wv[pl.ds(C * t + 16 * g, 16)]
                for j in range(16):
                    splat = jnp.full((16,), wg[j], jnp.float32)
                    i = 16 * g + j
                    for m in range(8):
                        rows_b[i, pl.ds(16 * m, 16)] = (
                            rows_b[i, pl.ds(16 * m, 16)] * splat)

            @pl.when(t > 0)
            def _():
                pltpu.make_async_copy(_dst_slice(t), dstv.at[b], sd[b]).wait()

            _issue_scatter(t, rows_b, dstv.at[b], ss[b], sw[b])
        return carry

    lax.fori_loop(0, NK // 4, _quad, 0)

    _wait_scatter(NK - 1, rows1, dstv.at[1], ss1, sw1)

    plsc.subcore_barrier()

    @pl.when(cid == 0)
    def _():
        pltpu.sync_copy(csh.at[pl.ds(base_n, ROWS_PER_SUB)],
                        clo_hbm.at[pl.ds(base_n, ROWS_PER_SUB)])

    @pl.when(cid == 1)
    def _():
        pltpu.sync_copy(csh.at[pl.ds(base_n, ROWS_PER_SUB)],
                        chi_hbm.at[pl.ds(base_n, ROWS_PER_SUB)])

    @pl.when(cid == 0)
    def _():
        pltpu.sync_copy(ssh.at[pl.ds(base_n, ROWS_PER_SUB)],
                        ssum_hbm.at[pl.ds(base_n, ROWS_PER_SUB)])


def _sc_call(src, dst, logits, hvlo, hvhi):
    mesh = plsc.VectorSubcoreMesh(core_axis_name="c", subcore_axis_name="s",
                                  num_cores=NCORE, num_subcores=NSUB)
    f = pl.kernel(
        _sc_body,
        out_type=[
            jax.ShapeDtypeStruct((NP, DH), jnp.float32),
            jax.ShapeDtypeStruct((NP, DH), jnp.float32),
            jax.ShapeDtypeStruct((NP,), jnp.float32),
        ],
        mesh=mesh,
        scratch_types=[
            pltpu.VMEM((4, C), jnp.int32),
            pltpu.VMEM((EPS,), jnp.float32),
            pltpu.VMEM((2, C), jnp.int32),
            pltpu.VMEM((C, DH), jnp.float32),
            pltpu.VMEM((C, DH), jnp.float32),
            pltpu.VMEM_SHARED((NP, DH), jnp.float32),
            pltpu.VMEM_SHARED((NP,), jnp.float32),
        ] + [pltpu.SemaphoreType.DMA] * 12,
    )
    return f(src, dst, logits, hvlo, hvhi)


def _dense_b_body(clo_ref, chi_ref, ssum_ref, gh_ref, h_ref, wih_ref, bih_ref,
                  out_ref):
    s = ssum_ref[...]
    s = jnp.where(s > 0.0, s, 1.0)
    c = jnp.concatenate([clo_ref[...], chi_ref[...]], axis=1) / s
    ctx = jnp.where(c > 0.0, c, jnp.exp(jnp.minimum(c, 0.0)) - 1.0)
    gi = lax.dot_general(ctx, wih_ref[...], (((1,), (1,)), ((), ())),
                         preferred_element_type=jnp.float32) + bih_ref[...]
    gh = gh_ref[...]
    h = h_ref[...]
    r = jax.nn.sigmoid(gi[:, :D] + gh[:, :D])
    z = jax.nn.sigmoid(gi[:, D:2 * D] + gh[:, D:2 * D])
    n = jnp.tanh(gi[:, 2 * D:] + r * gh[:, 2 * D:])
    out_ref[...] = jnp.maximum((1.0 - z) * n + z * h, 0.0)


def _dense_b(clo, chi, ssum2, gh, h, wih, bih):
    bn = 1000
    grid = (N // bn,)
    return pl.pallas_call(
        _dense_b_body,
        grid=grid,
        in_specs=[
            pl.BlockSpec((bn, DH), lambda i: (i, 0)),
            pl.BlockSpec((bn, DH), lambda i: (i, 0)),
            pl.BlockSpec((bn, 1), lambda i: (i, 0)),
            pl.BlockSpec((bn, 3 * D), lambda i: (i, 0)),
            pl.BlockSpec((bn, D), lambda i: (i, 0)),
            pl.BlockSpec((3 * D, D), lambda i: (0, 0)),
            pl.BlockSpec((1, 3 * D), lambda i: (0, 0)),
        ],
        out_specs=pl.BlockSpec((bn, D), lambda i: (i, 0)),
        out_shape=jax.ShapeDtypeStruct((N, D), jnp.float32),
    )(clo, chi, ssum2, gh, h, wih, bih)


def kernel(edge_index, edge_logits, node_feats, W_p, b_p, W_ih, W_hh, b_ih, b_hh):
    pad = EP - E
    pad_iota = jnp.arange(pad, dtype=jnp.int32)
    src = jnp.concatenate([edge_index[0], pad_iota % N])
    dst = jnp.concatenate([edge_index[1], N + pad_iota % (NP - N)])
    logits = jnp.concatenate(
        [edge_logits[:, 0], jnp.full((pad,), -jnp.inf, jnp.float32)])
    hvlo, hvhi = _dense_a1(node_feats, W_p, b_p.reshape(1, H))
    clo, chi, ssum = _sc_call(src, dst, logits, hvlo, hvhi)
    gh = _dense_a2(node_feats, W_hh, b_hh.reshape(1, 3 * D))
    return _dense_b(clo, chi, ssum.reshape(NP, 1), gh, node_feats,
                    W_ih, b_ih.reshape(1, 3 * D))
